# baseline (speedup 1.0000x reference)
import functools

import jax
import jax.numpy as jnp
import numpy as np
from jax import lax
from jax.experimental import pallas as pl
from jax.experimental.pallas import tpu as pltpu
from jax.experimental.pallas import tpu_sc as plsc

D_MODEL = 256
N_HEADS = 8
N_LEVELS = 3
N_POINTS = 4
HEAD_DIM = D_MODEL // N_HEADS
_LVL_HW = ((64, 64), (32, 32), (16, 16))
_LVL_OFF = (0, 4096, 5120)
S_TOTAL = 5376

_ROW_BLK = 512

_SEG = np.kron(np.eye(N_HEADS, dtype=np.float32),
               np.ones((N_LEVELS * N_POINTS, N_LEVELS * N_POINTS), np.float32))

_K_OF_T = np.arange(384) // 4
_C_OF_T = np.arange(384) % 4
_H_OF_K = _K_OF_T // 12
_LVL_OF_T = (_K_OF_T % 12) // N_POINTS
_P_OF_T = _K_OF_T % N_POINTS
_PI_K = (_H_OF_K * 12 + _P_OF_T * N_LEVELS + _LVL_OF_T).astype(np.int32)
_REPX = _K_OF_T.astype(np.int32)
_WL384 = np.array([_LVL_HW[l][1] for l in _LVL_OF_T], np.float32)
_HL384 = np.array([_LVL_HW[l][0] for l in _LVL_OF_T], np.float32)
_OFF4384 = (np.array([_LVL_OFF[l] * 4 for l in _LVL_OF_T], np.float32)
            + (_H_OF_K % 4).astype(np.float32))
_DY384 = (_C_OF_T // 2).astype(np.float32)
_DX384 = (_C_OF_T % 2).astype(np.float32)
_SEG384 = np.zeros((384, 384), np.float32)
for _t in range(384):
    for _t2 in range(384):
        if _C_OF_T[_t2] == 0 and _H_OF_K[_t2] == _H_OF_K[_t]:
            _SEG384[_t2, _t] = 1.0
_E3 = np.zeros((N_LEVELS, 384), np.float32)
for _t in range(384):
    _E3[_LVL_OF_T[_t], _t] = 1.0

_COLPERM256 = np.concatenate(
    [h * 32 + np.arange(32).reshape(2, 16).T.reshape(-1)
     for h in range(N_HEADS)]).astype(np.int32)


def _proj_kernel(q_ref, x_ref, rp_ref, wv_ref, bv_ref, wsox_ref, bsox_ref,
                 wsoy_ref, bsoy_ref, waw_ref, baw_ref, seg_ref, wl_ref,
                 hl_ref, off_ref, dy_ref, dx_ref, m3_ref, vlo_ref, vhi_ref,
                 idx_ref, wt_ref, *, n_rows_b):
    q = q_ref[...]
    v = jnp.dot(x_ref[...].astype(jnp.bfloat16),
                wv_ref[...].astype(jnp.bfloat16),
                preferred_element_type=jnp.float32) + bv_ref[...]
    v = v.astype(jnp.bfloat16)
    vlo_ref[...] = v[:, :128]
    vhi_ref[...] = v[:, 128:]
    so_x = jnp.dot(q, wsox_ref[...],
                   preferred_element_type=jnp.float32) + bsox_ref[...]
    so_y = jnp.dot(q, wsoy_ref[...],
                   preferred_element_type=jnp.float32) + bsoy_ref[...]
    t = jnp.dot(q.astype(jnp.bfloat16), waw_ref[...].astype(jnp.bfloat16),
                preferred_element_type=jnp.float32) + baw_ref[...]
    t = t - jnp.max(t, axis=-1, keepdims=True)
    e = jnp.exp(t)
    s = jnp.dot(e.astype(jnp.bfloat16), seg_ref[...].astype(jnp.bfloat16),
                preferred_element_type=jnp.float32)
    aw = e / s

    rp = rp_ref[...]
    rpx = (rp[:, 0:1] * m3_ref[0:1, :] + rp[:, 2:3] * m3_ref[1:2, :]
           + rp[:, 4:5] * m3_ref[2:3, :])
    rpy = (rp[:, 1:2] * m3_ref[0:1, :] + rp[:, 3:4] * m3_ref[1:2, :]
           + rp[:, 5:6] * m3_ref[2:3, :])
    wl = wl_ref[...]
    hl = hl_ref[...]
    dy = dy_ref[...]
    dx = dx_ref[...]
    px = rpx * wl + so_x - 0.5
    py = rpy * hl + so_y - 0.5
    x0 = jnp.floor(px)
    y0 = jnp.floor(py)
    fx = px - x0
    fy = py - y0
    cy = y0 + dy
    cx = x0 + dx
    wy = (1.0 - dy) + (2.0 * dy - 1.0) * fy
    wx = (1.0 - dx) + (2.0 * dx - 1.0) * fx
    valid = ((cx >= 0.0) & (cx <= wl - 1.0)
             & (cy >= 0.0) & (cy <= hl - 1.0)).astype(jnp.float32)
    cyc = jnp.clip(cy, 0.0, hl - 1.0)
    cxc = jnp.clip(cx, 0.0, wl - 1.0)
    row_i = (lax.broadcasted_iota(jnp.int32, px.shape, 0)
             + pl.program_id(0) * px.shape[0])
    b_col = jnp.where(row_i >= n_rows_b, float(S_TOTAL * 4), 0.0)
    idx_ref[...] = (b_col + off_ref[...]
                    + (cyc * wl + cxc) * 4.0).astype(jnp.int32)
    wt_ref[...] = aw * wx * wy * valid


def _projections(q2, x2, rp2, W_v, b_v, W_so, b_so, W_aw, b_aw):
    n_rows = q2.shape[0]
    grid = (n_rows // _ROW_BLK,)
    row_spec = pl.BlockSpec((_ROW_BLK, D_MODEL), lambda i: (i, 0))
    full = lambda a: pl.BlockSpec(a.shape, lambda i: (0,) * a.ndim)
    wv_t = W_v.T[:, jnp.asarray(_COLPERM256)]
    b_v = b_v[jnp.asarray(_COLPERM256)]
    wso_t = W_so.T
    repx = jnp.asarray(_REPX)
    repa = jnp.asarray(_PI_K)
    wsox = jnp.take(wso_t, 2 * repx, axis=1)
    bsox = jnp.take(b_so, 2 * repx)
    wsoy = jnp.take(wso_t, 2 * repx + 1, axis=1)
    bsoy = jnp.take(b_so, 2 * repx + 1)
    waw = jnp.take(W_aw.T, repa, axis=1)
    baw = jnp.take(b_aw, repa)
    seg = jnp.asarray(_SEG384)
    wl = jnp.asarray(_WL384)[None]
    hl = jnp.asarray(_HL384)[None]
    off4 = jnp.asarray(_OFF4384)[None]
    dy = jnp.asarray(_DY384)[None]
    dx = jnp.asarray(_DX384)[None]
    m3 = jnp.asarray(_E3)
    lane_spec = pl.BlockSpec((_ROW_BLK, 384), lambda i: (i, 0))
    return pl.pallas_call(
        functools.partial(_proj_kernel, n_rows_b=n_rows // 2),
        grid=grid,
        in_specs=[row_spec, row_spec,
                  pl.BlockSpec((_ROW_BLK, 2 * N_LEVELS), lambda i: (i, 0)),
                  full(wv_t), full(b_v[None]), full(wsox), full(bsox[None]),
                  full(wsoy), full(bsoy[None]), full(waw), full(baw[None]),
                  full(seg), full(wl), full(hl), full(off4), full(dy),
                  full(dx), full(m3)],
        out_specs=[
            pl.BlockSpec((_ROW_BLK, 128), lambda i: (i, 0)),
            pl.BlockSpec((_ROW_BLK, 128), lambda i: (i, 0)),
            lane_spec, lane_spec,
        ],
        out_shape=[
            jax.ShapeDtypeStruct((n_rows, 128), jnp.bfloat16),
            jax.ShapeDtypeStruct((n_rows, 128), jnp.bfloat16),
            jax.ShapeDtypeStruct((n_rows, 384), jnp.int32),
            jax.ShapeDtypeStruct((n_rows, 384), jnp.float32),
        ],
    )(q2, x2, rp2, wv_t, b_v[None], wsox, bsox[None], wsoy, bsoy[None], waw,
      baw[None], seg, wl, hl, off4, dy, dx, m3)


def _out_proj_kernel(a_ref, b_ref, wa_ref, wb_ref, bias_ref, o_ref):
    o_ref[...] = (jnp.dot(a_ref[...], wa_ref[...],
                          preferred_element_type=jnp.float32)
                  + jnp.dot(b_ref[...], wb_ref[...],
                            preferred_element_type=jnp.float32)
                  + bias_ref[...])


def _out_projection(xa, xb, W_o, b_o):
    n_rows = xa.shape[0]
    grid = (n_rows // _ROW_BLK,)
    row_spec = pl.BlockSpec((_ROW_BLK, 128), lambda i: (i, 0))
    wo_t = W_o.T
    wa = wo_t[:128]
    wb = wo_t[128:]
    return pl.pallas_call(
        _out_proj_kernel,
        grid=grid,
        in_specs=[row_spec, row_spec,
                  pl.BlockSpec(wa.shape, lambda i: (0, 0)),
                  pl.BlockSpec(wb.shape, lambda i: (0, 0)),
                  pl.BlockSpec((1, D_MODEL), lambda i: (0, 0))],
        out_specs=pl.BlockSpec((_ROW_BLK, D_MODEL), lambda i: (i, 0)),
        out_shape=jax.ShapeDtypeStruct((n_rows, D_MODEL), jnp.float32),
    )(xa, xb, wa, wb, b_o[None])


def _split_halves(idx384, wt384, nq):
    idx_a = idx384[:, :192].reshape(nq * 192 // 128, 128)
    idx_b = idx384[:, 192:].reshape(nq * 192 // 128, 128)
    wt_a = wt384[:, :192].reshape(nq * 192)
    wt_b = wt384[:, 192:].reshape(nq * 192)
    return idx_a, idx_b, wt_a, wt_b


_NC = 2
_NS = 16
_NW = _NC * _NS
_NQ = 2 * 5376
_QPW = _NQ // _NW
_CQ = 4
_NCHUNK = _QPW // _CQ
_NPTS = N_LEVELS * N_POINTS * 4
_HG = N_HEADS // 2
_RPG = _HG * _NPTS
_KG = _CQ * _RPG // 128
_CROWS = _CQ * _RPG


def _lane_bcast(vec, j):
    return lax.gather(
        vec, jnp.zeros((16, 1), jnp.int32) + j,
        lax.GatherDimensionNumbers(offset_dims=(), collapsed_slice_dims=(0,),
                                   start_index_map=(0,)),
        (1,), mode=lax.GatherScatterMode.PROMISE_IN_BOUNDS)


def _sc_gather_ws(vrows_a, vrows_b, idx_a, idx_b, wt_a, wt_b):
    mesh = plsc.VectorSubcoreMesh(core_axis_name="c", subcore_axis_name="s")

    @functools.partial(
        pl.kernel,
        mesh=mesh,
        compiler_params=pltpu.CompilerParams(use_tc_tiling_on_sc=False,
                                             needs_layout_passes=False),
        out_type=[jax.ShapeDtypeStruct((_NQ, 128), jnp.float32),
                  jax.ShapeDtypeStruct((_NQ, 128), jnp.float32)],
        scratch_types=[
            pltpu.VMEM((3, 2 * _KG, 128), jnp.int32),
            pltpu.VMEM((3, 2 * _CROWS, HEAD_DIM), jnp.bfloat16),
            pltpu.VMEM((3, 2 * _CROWS), jnp.float32),
            pltpu.VMEM((3, _CQ, 128), jnp.float32),
            pltpu.VMEM((3, _CQ, 128), jnp.float32),
        ] + [pltpu.SemaphoreType.DMA] * 9,
    )
    def sc_kernel(va_hbm, vb_hbm, idxa_hbm, idxb_hbm, wta_hbm, wtb_hbm,
                  outa_hbm, outb_hbm, idx_v, rows_v, wt_v, outa_v, outb_v,
                  st0, st1, st2, g0, g1, g2, o0, o1, o2):
        wid = lax.axis_index("s") * _NC + lax.axis_index("c")
        q0w = wid * _QPW
        st = (st0, st1, st2)
        gs = (g0, g1, g2)
        os = (o0, o1, o2)

        def fire_stage(c, b):
            row0 = q0w + c * _CQ
            ir0 = (row0 * 3) // 2
            pltpu.async_copy(idxa_hbm.at[pl.ds(ir0, _KG)],
                             idx_v.at[b, pl.ds(0, _KG)], st[b])
            pltpu.async_copy(idxb_hbm.at[pl.ds(ir0, _KG)],
                             idx_v.at[b, pl.ds(_KG, _KG)], st[b])
            pltpu.async_copy(wta_hbm.at[pl.ds(row0 * _RPG, _CROWS)],
                             wt_v.at[b, pl.ds(0, _CROWS)], st[b])
            pltpu.async_copy(wtb_hbm.at[pl.ds(row0 * _RPG, _CROWS)],
                             wt_v.at[b, pl.ds(_CROWS, _CROWS)], st[b])

        def drain_stage(b):
            for d in (idx_v.at[b, pl.ds(0, _KG)], idx_v.at[b, pl.ds(_KG, _KG)]):
                pltpu.make_async_copy(idxa_hbm.at[pl.ds(0, _KG)], d,
                                      st[b]).wait()
            for d in (wt_v.at[b, pl.ds(0, _CROWS)],
                      wt_v.at[b, pl.ds(_CROWS, _CROWS)]):
                pltpu.make_async_copy(wta_hbm.at[pl.ds(0, _CROWS)], d,
                                      st[b]).wait()

        def fire_gathers(b):
            for j in range(_KG):
                pltpu.async_copy(va_hbm.at[idx_v.at[b, j]],
                                 rows_v.at[b, pl.ds(j * 128, 128)], gs[b])
            for j in range(_KG):
                pltpu.async_copy(vb_hbm.at[idx_v.at[b, _KG + j]],
                                 rows_v.at[b, pl.ds(_CROWS + j * 128, 128)],
                                 gs[b])

        def drain_gathers(b):
            for _ in range(2 * _KG):
                pltpu.make_async_copy(va_hbm.at[pl.ds(0, 128)],
                                      rows_v.at[b, pl.ds(0, 128)],
                                      gs[b]).wait()

        def drain_out(b):
            pltpu.make_async_copy(outa_hbm.at[pl.ds(0, _CQ)],
                                  outa_v.at[b], os[b]).wait()
            pltpu.make_async_copy(outb_hbm.at[pl.ds(0, _CQ)],
                                  outb_v.at[b], os[b]).wait()

        def compute(c, b):
            row0 = q0w + c * _CQ
            for goff, out_v in ((0, outa_v), (_CROWS, outb_v)):

                def pair_body(t, carry2, goff=goff, out_v=out_v):
                    qq = t // _HG
                    hh = t - qq * _HG
                    base = goff + qq * _RPG + hh * _NPTS
                    acc0 = jnp.zeros((16,), jnp.float32)
                    acc1 = jnp.zeros((16,), jnp.float32)
                    for j0 in range(0, _NPTS, 16):
                        w16 = wt_v[b, pl.ds(base + j0, 16)]
                        for j in range(16):
                            p = base + j0 + j
                            w = _lane_bcast(w16, j)
                            r0, r1 = plsc.unpack(
                                rows_v[b, p, :],
                                format=plsc.PackFormat.INTERLEAVED)
                            acc0 = acc0 + w * r0
                            acc1 = acc1 + w * r1
                    out_v[b, qq, pl.ds(hh * HEAD_DIM, 16)] = acc0
                    out_v[b, qq, pl.ds(hh * HEAD_DIM + 16, 16)] = acc1
                    return carry2

                lax.fori_loop(0, _CQ * _HG, pair_body, 0)
            pltpu.async_copy(outa_v.at[b], outa_hbm.at[pl.ds(row0, _CQ)],
                             os[b])
            pltpu.async_copy(outb_v.at[b], outb_hbm.at[pl.ds(row0, _CQ)],
                             os[b])

        fire_stage(0, 0)
        fire_stage(1, 1)
        drain_stage(0)
        fire_gathers(0)

        def body(i, carry):
            for k in range(3):
                cs = 3 * i + k
                bn = (k + 1) % 3
                bn2 = (k + 2) % 3

                @pl.when(cs + 1 < _NCHUNK)
                def _(cs=cs, bn=bn):
                    drain_stage(bn)
                    fire_gathers(bn)

                @pl.when(cs + 2 < _NCHUNK)
                def _(cs=cs, bn2=bn2):
                    fire_stage(cs + 2, bn2)

                drain_gathers(k)

                @pl.when(cs >= 3)
                def _(k=k):
                    drain_out(k)

                compute(cs, k)
            return carry

        lax.fori_loop(0, _NCHUNK // 3, body, 0)
        for b in range(3):
            drain_out(b)

    return sc_kernel(vrows_a, vrows_b, idx_a, idx_b, wt_a, wt_b)


def kernel(query, reference_points, input_flatten, W_so, b_so, W_aw, b_aw,
           W_v, b_v, W_o, b_o, input_spatial_shapes):
    B, Lq, _ = query.shape
    q2 = query.reshape(B * Lq, D_MODEL)
    x2 = input_flatten.reshape(B * S_TOTAL, D_MODEL)
    rp2 = reference_points.reshape(B * Lq, 2 * N_LEVELS)
    v_lo, v_hi, idx384, wt384 = _projections(q2, x2, rp2, W_v, b_v, W_so,
                                             b_so, W_aw, b_aw)
    idx_a, idx_b, wt_a, wt_b = _split_halves(idx384, wt384, B * Lq)

    vrows_a = v_lo.reshape(B * S_TOTAL * 4, HEAD_DIM)
    vrows_b = v_hi.reshape(B * S_TOTAL * 4, HEAD_DIM)
    out_a, out_b = _sc_gather_ws(vrows_a, vrows_b, idx_a, idx_b, wt_a, wt_b)
    out = _out_projection(out_a, out_b, W_o, b_o)
    return out.reshape(B, Lq, D_MODEL)


# speedup vs baseline: 1.1290x; 1.1290x over previous
import functools

import jax
import jax.numpy as jnp
import numpy as np
from jax import lax
from jax.experimental import pallas as pl
from jax.experimental.pallas import tpu as pltpu
from jax.experimental.pallas import tpu_sc as plsc

D_MODEL = 256
N_HEADS = 8
N_LEVELS = 3
N_POINTS = 4
HEAD_DIM = D_MODEL // N_HEADS
_LVL_HW = ((64, 64), (32, 32), (16, 16))
_LVL_OFF = (0, 4096, 5120)
S_TOTAL = 5376

_ROW_BLK = 512

_SEG = np.kron(np.eye(N_HEADS, dtype=np.float32),
               np.ones((N_LEVELS * N_POINTS, N_LEVELS * N_POINTS), np.float32))

_K_OF_T = np.arange(384) // 4
_C_OF_T = np.arange(384) % 4
_H_OF_K = _K_OF_T // 12
_LVL_OF_T = (_K_OF_T % 12) // N_POINTS
_P_OF_T = _K_OF_T % N_POINTS
_PI_K = (_H_OF_K * 12 + _P_OF_T * N_LEVELS + _LVL_OF_T).astype(np.int32)
_REPX = _K_OF_T.astype(np.int32)
_WL384 = np.array([_LVL_HW[l][1] for l in _LVL_OF_T], np.float32)
_HL384 = np.array([_LVL_HW[l][0] for l in _LVL_OF_T], np.float32)
_OFF4384 = (np.array([_LVL_OFF[l] * 4 for l in _LVL_OF_T], np.float32)
            + (_H_OF_K % 4).astype(np.float32))
_DY384 = (_C_OF_T // 2).astype(np.float32)
_DX384 = (_C_OF_T % 2).astype(np.float32)
_SEG384 = np.zeros((384, 384), np.float32)
for _t in range(384):
    for _t2 in range(384):
        if _C_OF_T[_t2] == 0 and _H_OF_K[_t2] == _H_OF_K[_t]:
            _SEG384[_t2, _t] = 1.0
_E3 = np.zeros((N_LEVELS, 384), np.float32)
for _t in range(384):
    _E3[_LVL_OF_T[_t], _t] = 1.0

_COLPERM256 = np.concatenate(
    [h * 32 + np.arange(32).reshape(2, 16).T.reshape(-1)
     for h in range(N_HEADS)]).astype(np.int32)


def _proj_kernel(q_ref, x_ref, rp_ref, wv_ref, bv_ref, wsox_ref, bsox_ref,
                 wsoy_ref, bsoy_ref, waw_ref, baw_ref, seg_ref, wl_ref,
                 hl_ref, off_ref, dy_ref, dx_ref, m3_ref, vlo_ref, vhi_ref,
                 idx_ref, wt_ref, *, n_rows_b):
    q = q_ref[...]
    v = jnp.dot(x_ref[...], wv_ref[...],
                preferred_element_type=jnp.float32) + bv_ref[...]
    v = v.astype(jnp.bfloat16)
    vlo_ref[...] = v[:, :128]
    vhi_ref[...] = v[:, 128:]
    so_x = jnp.dot(q, wsox_ref[...],
                   preferred_element_type=jnp.float32) + bsox_ref[...]
    so_y = jnp.dot(q, wsoy_ref[...],
                   preferred_element_type=jnp.float32) + bsoy_ref[...]
    t = jnp.dot(q, waw_ref[...], preferred_element_type=jnp.float32) + baw_ref[...]
    t = t - jnp.max(t, axis=-1, keepdims=True)
    e = jnp.exp(t)
    s = jnp.dot(e, seg_ref[...], preferred_element_type=jnp.float32)
    aw = e / s

    rp = rp_ref[...]
    rpx = (rp[:, 0:1] * m3_ref[0:1, :] + rp[:, 2:3] * m3_ref[1:2, :]
           + rp[:, 4:5] * m3_ref[2:3, :])
    rpy = (rp[:, 1:2] * m3_ref[0:1, :] + rp[:, 3:4] * m3_ref[1:2, :]
           + rp[:, 5:6] * m3_ref[2:3, :])
    wl = wl_ref[...]
    hl = hl_ref[...]
    dy = dy_ref[...]
    dx = dx_ref[...]
    px = rpx * wl + so_x - 0.5
    py = rpy * hl + so_y - 0.5
    x0 = jnp.floor(px)
    y0 = jnp.floor(py)
    fx = px - x0
    fy = py - y0
    cy = y0 + dy
    cx = x0 + dx
    wy = (1.0 - dy) + (2.0 * dy - 1.0) * fy
    wx = (1.0 - dx) + (2.0 * dx - 1.0) * fx
    valid = ((cx >= 0.0) & (cx <= wl - 1.0)
             & (cy >= 0.0) & (cy <= hl - 1.0)).astype(jnp.float32)
    cyc = jnp.clip(cy, 0.0, hl - 1.0)
    cxc = jnp.clip(cx, 0.0, wl - 1.0)
    row_i = (lax.broadcasted_iota(jnp.int32, px.shape, 0)
             + pl.program_id(0) * px.shape[0])
    b_col = jnp.where(row_i >= n_rows_b, float(S_TOTAL * 4), 0.0)
    idx_ref[...] = (b_col + off_ref[...]
                    + (cyc * wl + cxc) * 4.0).astype(jnp.int32)
    wt_ref[...] = aw * wx * wy * valid


def _projections(q2, x2, rp2, W_v, b_v, W_so, b_so, W_aw, b_aw):
    n_rows = q2.shape[0]
    grid = (n_rows // _ROW_BLK,)
    row_spec = pl.BlockSpec((_ROW_BLK, D_MODEL), lambda i: (i, 0))
    full = lambda a: pl.BlockSpec(a.shape, lambda i: (0,) * a.ndim)
    wv_t = W_v.T[:, jnp.asarray(_COLPERM256)]
    b_v = b_v[jnp.asarray(_COLPERM256)]
    wso_t = W_so.T
    repx = jnp.asarray(_REPX)
    repa = jnp.asarray(_PI_K)
    wsox = jnp.take(wso_t, 2 * repx, axis=1)
    bsox = jnp.take(b_so, 2 * repx)
    wsoy = jnp.take(wso_t, 2 * repx + 1, axis=1)
    bsoy = jnp.take(b_so, 2 * repx + 1)
    waw = jnp.take(W_aw.T, repa, axis=1)
    baw = jnp.take(b_aw, repa)
    seg = jnp.asarray(_SEG384)
    wl = jnp.asarray(_WL384)[None]
    hl = jnp.asarray(_HL384)[None]
    off4 = jnp.asarray(_OFF4384)[None]
    dy = jnp.asarray(_DY384)[None]
    dx = jnp.asarray(_DX384)[None]
    m3 = jnp.asarray(_E3)
    lane_spec = pl.BlockSpec((_ROW_BLK, 384), lambda i: (i, 0))
    return pl.pallas_call(
        functools.partial(_proj_kernel, n_rows_b=n_rows // 2),
        grid=grid,
        in_specs=[row_spec, row_spec,
                  pl.BlockSpec((_ROW_BLK, 2 * N_LEVELS), lambda i: (i, 0)),
                  full(wv_t), full(b_v[None]), full(wsox), full(bsox[None]),
                  full(wsoy), full(bsoy[None]), full(waw), full(baw[None]),
                  full(seg), full(wl), full(hl), full(off4), full(dy),
                  full(dx), full(m3)],
        out_specs=[
            pl.BlockSpec((_ROW_BLK, 128), lambda i: (i, 0)),
            pl.BlockSpec((_ROW_BLK, 128), lambda i: (i, 0)),
            lane_spec, lane_spec,
        ],
        out_shape=[
            jax.ShapeDtypeStruct((n_rows, 128), jnp.bfloat16),
            jax.ShapeDtypeStruct((n_rows, 128), jnp.bfloat16),
            jax.ShapeDtypeStruct((n_rows, 384), jnp.int32),
            jax.ShapeDtypeStruct((n_rows, 384), jnp.float32),
        ],
    )(q2, x2, rp2, wv_t, b_v[None], wsox, bsox[None], wsoy, bsoy[None], waw,
      baw[None], seg, wl, hl, off4, dy, dx, m3)


def _out_proj_kernel(a_ref, b_ref, wa_ref, wb_ref, bias_ref, o_ref):
    o_ref[...] = (jnp.dot(a_ref[...], wa_ref[...],
                          preferred_element_type=jnp.float32)
                  + jnp.dot(b_ref[...], wb_ref[...],
                            preferred_element_type=jnp.float32)
                  + bias_ref[...])


def _out_projection(xa, xb, W_o, b_o):
    n_rows = xa.shape[0]
    grid = (n_rows // _ROW_BLK,)
    row_spec = pl.BlockSpec((_ROW_BLK, 128), lambda i: (i, 0))
    wo_t = W_o.T
    wa = wo_t[:128]
    wb = wo_t[128:]
    return pl.pallas_call(
        _out_proj_kernel,
        grid=grid,
        in_specs=[row_spec, row_spec,
                  pl.BlockSpec(wa.shape, lambda i: (0, 0)),
                  pl.BlockSpec(wb.shape, lambda i: (0, 0)),
                  pl.BlockSpec((1, D_MODEL), lambda i: (0, 0))],
        out_specs=pl.BlockSpec((_ROW_BLK, D_MODEL), lambda i: (i, 0)),
        out_shape=jax.ShapeDtypeStruct((n_rows, D_MODEL), jnp.float32),
    )(xa, xb, wa, wb, b_o[None])


_NC = 2
_NS = 16
_NW = _NC * _NS
_NQ = 2 * 5376
_QPW = _NQ // _NW
_CQ = 4
_NCHUNK = _QPW // _CQ
_NPTS = N_LEVELS * N_POINTS * 4
_HG = N_HEADS // 2
_RPG = _HG * _NPTS
_KG = _CQ * _RPG // 128
_CROWS = _CQ * _RPG


def _lane_bcast(vec, j):
    return lax.gather(
        vec, jnp.zeros((16, 1), jnp.int32) + j,
        lax.GatherDimensionNumbers(offset_dims=(), collapsed_slice_dims=(0,),
                                   start_index_map=(0,)),
        (1,), mode=lax.GatherScatterMode.PROMISE_IN_BOUNDS)


def _sc_gather_ws(vrows_a, vrows_b, idx2, wtf):
    mesh = plsc.VectorSubcoreMesh(core_axis_name="c", subcore_axis_name="s")

    @functools.partial(
        pl.kernel,
        mesh=mesh,
        compiler_params=pltpu.CompilerParams(use_tc_tiling_on_sc=False,
                                             needs_layout_passes=False),
        out_type=[jax.ShapeDtypeStruct((_NQ, 128), jnp.float32),
                  jax.ShapeDtypeStruct((_NQ, 128), jnp.float32)],
        scratch_types=[
            pltpu.VMEM((3, 2 * _KG, 128), jnp.int32),
            pltpu.VMEM((3, 2 * _CROWS, HEAD_DIM), jnp.bfloat16),
            pltpu.VMEM((3, 2 * _CROWS), jnp.float32),
            pltpu.VMEM((3, _CQ, 128), jnp.float32),
            pltpu.VMEM((3, _CQ, 128), jnp.float32),
        ] + [pltpu.SemaphoreType.DMA] * 9,
    )
    def sc_kernel(va_hbm, vb_hbm, idx_hbm, wt_hbm,
                  outa_hbm, outb_hbm, idx_v, rows_v, wt_v, outa_v, outb_v,
                  st0, st1, st2, g0, g1, g2, o0, o1, o2):
        wid = lax.axis_index("s") * _NC + lax.axis_index("c")
        q0w = wid * _QPW
        st = (st0, st1, st2)
        gs = (g0, g1, g2)
        os = (o0, o1, o2)

        def fire_stage(c, b):
            row0 = q0w + c * _CQ
            pltpu.async_copy(idx_hbm.at[pl.ds(row0 * 3, 2 * _KG)],
                             idx_v.at[b], st[b])
            pltpu.async_copy(wt_hbm.at[pl.ds(row0 * 2 * _RPG, 2 * _CROWS)],
                             wt_v.at[b], st[b])

        def drain_stage(b):
            pltpu.make_async_copy(idx_hbm.at[pl.ds(0, 2 * _KG)],
                                  idx_v.at[b], st[b]).wait()
            pltpu.make_async_copy(wt_hbm.at[pl.ds(0, 2 * _CROWS)],
                                  wt_v.at[b], st[b]).wait()

        def fire_gathers(b):
            for q in range(_CQ):
                for g, src in ((0, va_hbm), (1, vb_hbm)):
                    for j in range(3):
                        e0 = q * 2 * _RPG + g * _RPG + j * 64
                        pltpu.async_copy(
                            src.at[idx_v.at[b, e0 // 128,
                                            pl.ds(e0 % 128, 64)]],
                            rows_v.at[b, pl.ds(e0, 64)], gs[b])

        def drain_gathers(b):
            for _ in range(6 * _CQ):
                pltpu.make_async_copy(va_hbm.at[pl.ds(0, 64)],
                                      rows_v.at[b, pl.ds(0, 64)],
                                      gs[b]).wait()

        def drain_out(b):
            pltpu.make_async_copy(outa_hbm.at[pl.ds(0, _CQ)],
                                  outa_v.at[b], os[b]).wait()
            pltpu.make_async_copy(outb_hbm.at[pl.ds(0, _CQ)],
                                  outb_v.at[b], os[b]).wait()

        def compute(c, b):
            row0 = q0w + c * _CQ
            for goff, out_v in ((0, outa_v), (_RPG, outb_v)):

                def pair_body(t, carry2, goff=goff, out_v=out_v):
                    qq = t // _HG
                    hh = t - qq * _HG
                    base = qq * 2 * _RPG + goff + hh * _NPTS
                    acc0 = jnp.zeros((16,), jnp.float32)
                    acc1 = jnp.zeros((16,), jnp.float32)
                    for j0 in range(0, _NPTS, 16):
                        w16 = wt_v[b, pl.ds(base + j0, 16)]
                        for j in range(16):
                            p = base + j0 + j
                            w = _lane_bcast(w16, j)
                            r0, r1 = plsc.unpack(
                                rows_v[b, p, :],
                                format=plsc.PackFormat.INTERLEAVED)
                            acc0 = acc0 + w * r0
                            acc1 = acc1 + w * r1
                    out_v[b, qq, pl.ds(hh * HEAD_DIM, 16)] = acc0
                    out_v[b, qq, pl.ds(hh * HEAD_DIM + 16, 16)] = acc1
                    return carry2

                lax.fori_loop(0, _CQ * _HG, pair_body, 0)
            pltpu.async_copy(outa_v.at[b], outa_hbm.at[pl.ds(row0, _CQ)],
                             os[b])
            pltpu.async_copy(outb_v.at[b], outb_hbm.at[pl.ds(row0, _CQ)],
                             os[b])

        fire_stage(0, 0)
        fire_stage(1, 1)
        drain_stage(0)
        fire_gathers(0)

        def body(i, carry):
            for k in range(3):
                cs = 3 * i + k
                bn = (k + 1) % 3
                bn2 = (k + 2) % 3

                @pl.when(cs + 1 < _NCHUNK)
                def _(cs=cs, bn=bn):
                    drain_stage(bn)
                    fire_gathers(bn)

                @pl.when(cs + 2 < _NCHUNK)
                def _(cs=cs, bn2=bn2):
                    fire_stage(cs + 2, bn2)

                drain_gathers(k)

                @pl.when(cs >= 3)
                def _(k=k):
                    drain_out(k)

                compute(cs, k)
            return carry

        lax.fori_loop(0, _NCHUNK // 3, body, 0)
        for b in range(3):
            drain_out(b)

    return sc_kernel(vrows_a, vrows_b, idx2, wtf)


def kernel(query, reference_points, input_flatten, W_so, b_so, W_aw, b_aw,
           W_v, b_v, W_o, b_o, input_spatial_shapes):
    B, Lq, _ = query.shape
    q2 = query.reshape(B * Lq, D_MODEL)
    x2 = input_flatten.reshape(B * S_TOTAL, D_MODEL)
    rp2 = reference_points.reshape(B * Lq, 2 * N_LEVELS)
    v_lo, v_hi, idx384, wt384 = _projections(q2, x2, rp2, W_v, b_v, W_so,
                                             b_so, W_aw, b_aw)
    idx2 = idx384.reshape(B * Lq * 3, 128)
    wtf = wt384.reshape(B * Lq * 384)

    vrows_a = v_lo.reshape(B * S_TOTAL * 4, HEAD_DIM)
    vrows_b = v_hi.reshape(B * S_TOTAL * 4, HEAD_DIM)
    out_a, out_b = _sc_gather_ws(vrows_a, vrows_b, idx2, wtf)
    out = _out_projection(out_a, out_b, W_o, b_o)
    return out.reshape(B, Lq, D_MODEL)
